# baseline (speedup 1.0000x reference)
import functools

import jax
import jax.numpy as jnp
from jax import lax
from jax.experimental import pallas as pl
from jax.experimental.pallas import tpu as pltpu
from jax.experimental.pallas import tpu_sc as plsc

NUM_TIME_STEPS = 1000
BATCH = 16384

_info = plsc.get_sparse_core_info()
_NC, _NS, _L = _info.num_cores, _info.num_subcores, _info.num_lanes
_NW = _NS
_B_PER_W = BATCH // _NW
_VECS = _B_PER_W // _L


@functools.partial(
    pl.kernel,
    mesh=plsc.VectorSubcoreMesh(core_axis_name="c", subcore_axis_name="s", num_cores=1),
    compiler_params=pltpu.CompilerParams(needs_layout_passes=False),
    out_type=(
        jax.ShapeDtypeStruct((BATCH,), jnp.float32),
        jax.ShapeDtypeStruct((BATCH,), jnp.float32),
    ),
    scratch_types=[
        pltpu.VMEM((NUM_TIME_STEPS,), jnp.float32),
        pltpu.VMEM((NUM_TIME_STEPS,), jnp.float32),
        pltpu.VMEM((_B_PER_W,), jnp.int32),
        pltpu.VMEM((_B_PER_W,), jnp.float32),
        pltpu.VMEM((_B_PER_W,), jnp.float32),
        pltpu.SemaphoreType.DMA,
        pltpu.SemaphoreType.DMA,
        pltpu.SemaphoreType.DMA,
    ],
)
def _ddpm_gather(alpha_hbm, beta_hbm, ts_hbm, a_out_hbm, b_out_hbm,
                 alpha_v, beta_v, idx_v, a_v, b_v, sem, sem_b, sem_out):
    wid = lax.axis_index("s")
    base = wid * _B_PER_W
    cp_a = pltpu.async_copy(alpha_hbm, alpha_v, sem)
    cp_i = pltpu.async_copy(ts_hbm.at[pl.ds(base, _B_PER_W)], idx_v, sem)
    cp_b = pltpu.async_copy(beta_hbm, beta_v, sem_b)
    cp_a.wait()
    cp_i.wait()

    def a_body(i, carry):
        sl = pl.ds(i * _L, _L)
        a_v[sl] = plsc.load_gather(alpha_v, [idx_v[sl]])
        return carry

    lax.fori_loop(0, _VECS, a_body, 0, unroll=4)

    out_a = pltpu.async_copy(a_v, a_out_hbm.at[pl.ds(base, _B_PER_W)], sem_out)
    cp_b.wait()

    def b_body(i, carry):
        sl = pl.ds(i * _L, _L)
        b_v[sl] = plsc.load_gather(beta_v, [idx_v[sl]])
        return carry

    lax.fori_loop(0, _VECS, b_body, 0, unroll=4)

    out_b = pltpu.async_copy(b_v, b_out_hbm.at[pl.ds(base, _B_PER_W)], sem_out)
    out_a.wait()
    out_b.wait()


def kernel(alpha, beta, timestep):
    return _ddpm_gather(alpha, beta, timestep)


# speedup vs baseline: 1.0351x; 1.0351x over previous
import functools

import jax
import jax.numpy as jnp
from jax import lax
from jax.experimental import pallas as pl
from jax.experimental.pallas import tpu as pltpu
from jax.experimental.pallas import tpu_sc as plsc

NUM_TIME_STEPS = 1000
BATCH = 16384

_info = plsc.get_sparse_core_info()
_NC, _NS, _L = _info.num_cores, _info.num_subcores, _info.num_lanes
_NW = _NS
_B_PER_W = BATCH // _NW
_VECS = _B_PER_W // _L


@functools.partial(
    pl.kernel,
    mesh=plsc.VectorSubcoreMesh(core_axis_name="c", subcore_axis_name="s", num_cores=1),
    compiler_params=pltpu.CompilerParams(needs_layout_passes=False),
    out_type=(
        jax.ShapeDtypeStruct((BATCH,), jnp.float32),
        jax.ShapeDtypeStruct((BATCH,), jnp.float32),
    ),
    scratch_types=[
        pltpu.VMEM((NUM_TIME_STEPS,), jnp.float32),
        pltpu.VMEM((NUM_TIME_STEPS,), jnp.float32),
        pltpu.VMEM((_B_PER_W,), jnp.int32),
        pltpu.VMEM((_B_PER_W,), jnp.float32),
        pltpu.VMEM((_B_PER_W,), jnp.float32),
        pltpu.SemaphoreType.DMA,
        pltpu.SemaphoreType.DMA,
        pltpu.SemaphoreType.DMA,
    ],
)
def _ddpm_gather(alpha_hbm, beta_hbm, ts_hbm, a_out_hbm, b_out_hbm,
                 alpha_v, beta_v, idx_v, a_v, b_v, sem, sem_b, sem_out):
    wid = lax.axis_index("s")
    base = wid * _B_PER_W
    cp_a = pltpu.async_copy(alpha_hbm, alpha_v, sem)
    cp_i = pltpu.async_copy(ts_hbm.at[pl.ds(base, _B_PER_W)], idx_v, sem)
    cp_b = pltpu.async_copy(beta_hbm, beta_v, sem_b)
    cp_a.wait()
    cp_i.wait()

    @plsc.parallel_loop(0, _B_PER_W, _L, unroll=2)
    def a_body(i):
        sl = pl.ds(i, _L)
        a_v[sl] = plsc.load_gather(alpha_v, [idx_v[sl]])

    out_a = pltpu.async_copy(a_v, a_out_hbm.at[pl.ds(base, _B_PER_W)], sem_out)
    cp_b.wait()

    @plsc.parallel_loop(0, _B_PER_W, _L, unroll=2)
    def b_body(i):
        sl = pl.ds(i, _L)
        b_v[sl] = plsc.load_gather(beta_v, [idx_v[sl]])

    out_b = pltpu.async_copy(b_v, b_out_hbm.at[pl.ds(base, _B_PER_W)], sem_out)
    out_a.wait()
    out_b.wait()


def kernel(alpha, beta, timestep):
    return _ddpm_gather(alpha, beta, timestep)


# speedup vs baseline: 1.0446x; 1.0092x over previous
import functools

import jax
import jax.numpy as jnp
from jax import lax
from jax.experimental import pallas as pl
from jax.experimental.pallas import tpu as pltpu
from jax.experimental.pallas import tpu_sc as plsc

NUM_TIME_STEPS = 1000
BATCH = 16384

_info = plsc.get_sparse_core_info()
_NC, _NS, _L = _info.num_cores, _info.num_subcores, _info.num_lanes
_NW = _NS
_B_PER_W = BATCH // _NW
_VECS = _B_PER_W // _L


@functools.partial(
    pl.kernel,
    mesh=plsc.VectorSubcoreMesh(core_axis_name="c", subcore_axis_name="s", num_cores=1),
    compiler_params=pltpu.CompilerParams(needs_layout_passes=False),
    out_type=(
        jax.ShapeDtypeStruct((BATCH,), jnp.float32),
        jax.ShapeDtypeStruct((BATCH,), jnp.float32),
    ),
    scratch_types=[
        pltpu.VMEM((NUM_TIME_STEPS,), jnp.float32),
        pltpu.VMEM((NUM_TIME_STEPS,), jnp.float32),
        pltpu.VMEM((_B_PER_W,), jnp.int32),
        pltpu.VMEM((_B_PER_W,), jnp.float32),
        pltpu.VMEM((_B_PER_W,), jnp.float32),
        pltpu.SemaphoreType.DMA,
        pltpu.SemaphoreType.DMA,
        pltpu.SemaphoreType.DMA,
    ],
)
def _ddpm_gather(alpha_hbm, beta_hbm, ts_hbm, a_out_hbm, b_out_hbm,
                 alpha_v, beta_v, idx_v, a_v, b_v, sem, sem_b, sem_out):
    wid = lax.axis_index("s")
    base = wid * _B_PER_W
    cp_a = pltpu.async_copy(alpha_hbm, alpha_v, sem)
    cp_i = pltpu.async_copy(ts_hbm.at[pl.ds(base, _B_PER_W)], idx_v, sem)
    cp_b = pltpu.async_copy(beta_hbm, beta_v, sem_b)
    cp_a.wait()
    cp_i.wait()

    @plsc.parallel_loop(0, _B_PER_W, _L, unroll=4)
    def a_body(i):
        sl = pl.ds(i, _L)
        a_v[sl] = plsc.load_gather(alpha_v, [idx_v[sl]])

    out_a = pltpu.async_copy(a_v, a_out_hbm.at[pl.ds(base, _B_PER_W)], sem_out)
    cp_b.wait()

    @plsc.parallel_loop(0, _B_PER_W, _L, unroll=4)
    def b_body(i):
        sl = pl.ds(i, _L)
        b_v[sl] = plsc.load_gather(beta_v, [idx_v[sl]])

    out_b = pltpu.async_copy(b_v, b_out_hbm.at[pl.ds(base, _B_PER_W)], sem_out)
    out_a.wait()
    out_b.wait()


def kernel(alpha, beta, timestep):
    return _ddpm_gather(alpha, beta, timestep)


# speedup vs baseline: 1.0450x; 1.0004x over previous
import functools

import jax
import jax.numpy as jnp
from jax import lax
from jax.experimental import pallas as pl
from jax.experimental.pallas import tpu as pltpu
from jax.experimental.pallas import tpu_sc as plsc

NUM_TIME_STEPS = 1000
BATCH = 16384

_info = plsc.get_sparse_core_info()
_NC, _NS, _L = _info.num_cores, _info.num_subcores, _info.num_lanes
_NW = _NS
_B_PER_W = BATCH // _NW
_VECS = _B_PER_W // _L


@functools.partial(
    pl.kernel,
    mesh=plsc.VectorSubcoreMesh(core_axis_name="c", subcore_axis_name="s", num_cores=1),
    compiler_params=pltpu.CompilerParams(needs_layout_passes=False),
    out_type=(
        jax.ShapeDtypeStruct((BATCH,), jnp.float32),
        jax.ShapeDtypeStruct((BATCH,), jnp.float32),
    ),
    scratch_types=[
        pltpu.VMEM((NUM_TIME_STEPS,), jnp.float32),
        pltpu.VMEM((NUM_TIME_STEPS,), jnp.float32),
        pltpu.VMEM((_B_PER_W,), jnp.int32),
        pltpu.VMEM((_B_PER_W,), jnp.float32),
        pltpu.VMEM((_B_PER_W,), jnp.float32),
        pltpu.SemaphoreType.DMA,
        pltpu.SemaphoreType.DMA,
        pltpu.SemaphoreType.DMA,
    ],
)
def _ddpm_gather(alpha_hbm, beta_hbm, ts_hbm, a_out_hbm, b_out_hbm,
                 alpha_v, beta_v, idx_v, a_v, b_v, sem, sem_b, sem_out):
    wid = lax.axis_index("s")
    base = wid * _B_PER_W
    cp_a = pltpu.async_copy(alpha_hbm, alpha_v, sem)
    cp_i = pltpu.async_copy(ts_hbm.at[pl.ds(base, _B_PER_W)], idx_v, sem)
    cp_b = pltpu.async_copy(beta_hbm, beta_v, sem_b)
    cp_a.wait()
    cp_i.wait()

    @plsc.parallel_loop(0, _B_PER_W, _L, unroll=8)
    def a_body(i):
        sl = pl.ds(i, _L)
        a_v[sl] = plsc.load_gather(alpha_v, [idx_v[sl]])

    out_a = pltpu.async_copy(a_v, a_out_hbm.at[pl.ds(base, _B_PER_W)], sem_out)
    cp_b.wait()

    @plsc.parallel_loop(0, _B_PER_W, _L, unroll=8)
    def b_body(i):
        sl = pl.ds(i, _L)
        b_v[sl] = plsc.load_gather(beta_v, [idx_v[sl]])

    out_b = pltpu.async_copy(b_v, b_out_hbm.at[pl.ds(base, _B_PER_W)], sem_out)
    out_a.wait()
    out_b.wait()


def kernel(alpha, beta, timestep):
    return _ddpm_gather(alpha, beta, timestep)
